# Initial kernel scaffold; baseline (speedup 1.0000x reference)
import jax
import jax.numpy as jnp
from jax.experimental import pallas as pl


def kernel(x, a, c, e, edge_index, batch, t, params):
    raise NotImplementedError("write your pallas kernel here")


# speedup vs baseline: 1.0892x; 1.0892x over previous
import functools

import jax
import jax.numpy as jnp
from jax import lax
from jax.experimental import pallas as pl
from jax.experimental.pallas import tpu as pltpu
from jax.experimental.pallas import tpu_sc as plsc

N = 10000
E = 320000
G = 32
D = 128
CP = 16
BE = 2000

_f32 = jnp.float32


def _embed_body(a_ref, c_ref, b_ref, t_ref, atom_ref, charge_ref,
                wn_ref, bn_ref, wt_ref, bt_ref, h_ref):
    af = a_ref[...]
    cf = c_ref[...]
    bf = b_ref[...]
    aoh = (af == lax.broadcasted_iota(jnp.int32, (N, 16), 1).astype(_f32)).astype(_f32)
    coh = (cf == lax.broadcasted_iota(jnp.int32, (N, 8), 1).astype(_f32)).astype(_f32)
    boh = (bf == lax.broadcasted_iota(jnp.int32, (N, G), 1).astype(_f32)).astype(_f32)
    ones = jnp.ones((N, 1), _f32)
    counts = lax.dot_general(boh, ones, (((0,), (0,)), ((), ())),
                             preferred_element_type=_f32)
    n_tbl = jnp.log1p(counts) @ wn_ref[...] + bn_ref[...]
    t_tbl = t_ref[...] @ wt_ref[...] + bt_ref[...]
    h_ref[...] = jnp.concatenate(
        [aoh @ atom_ref[...], coh @ charge_ref[...],
         boh @ n_tbl, boh @ t_tbl], axis=1)


def _embed_call(a_f, c_f, b_f, t_col, p):
    return pl.pallas_call(
        _embed_body,
        out_shape=jax.ShapeDtypeStruct((N, D), _f32),
    )(a_f, c_f, b_f, t_col, p['atom_emb'], p['charge_emb'],
      p['Wn'], p['bn'].reshape(1, -1), p['Wt'], p['bt'].reshape(1, -1))


def _edge_body(hr_ref, hc_ref, cr_ref, cc_ref, e_ref,
               A_ref, B_ref, wc_ref, wd_ref, eemb_ref, be1_ref,
               W2_ref, be2_ref, wx_ref, bx_ref,
               m2_ref, wdout_ref):
    eoh = (e_ref[...] == lax.broadcasted_iota(jnp.int32, (BE, 5), 1).astype(_f32)).astype(_f32)
    te = eemb_ref[...] @ wd_ref[...]
    diff = cr_ref[...] - cc_ref[...]
    d2 = jnp.sum(diff * diff, axis=1, keepdims=True)
    z1 = (jnp.dot(hr_ref[...], A_ref[...], preferred_element_type=_f32)
          + jnp.dot(hc_ref[...], B_ref[...], preferred_element_type=_f32)
          + d2 * wc_ref[...] + eoh @ te + be1_ref[...])
    m1 = z1 * jax.nn.sigmoid(z1)
    z2 = jnp.dot(m1, W2_ref[...], preferred_element_type=_f32) + be2_ref[...]
    m2 = z2 * jax.nn.sigmoid(z2)
    w = jnp.sum(m2 * wx_ref[...], axis=1, keepdims=True) + bx_ref[...]
    m2_ref[...] = m2
    lane = lax.broadcasted_iota(jnp.int32, (BE, CP), 1)
    wdout_ref[...] = diff * w + (lane == 3).astype(_f32)


def _edge_call(hr, hc, cr, cc, e_f, lp, edge_emb):
    nb = E // BE
    A = lp['We1'][0:D]
    B = lp['We1'][D:2 * D]
    wc = lp['We1'][2 * D:2 * D + 1]
    Wd = lp['We1'][2 * D + 1:]
    eb = lambda w: pl.BlockSpec((BE, w), lambda i: (i, 0))
    fb = lambda arr: pl.BlockSpec(arr.shape, lambda i: (0,) * arr.ndim)
    args = (hr, hc, cr, cc, e_f, A, B, wc, Wd, edge_emb,
            lp['be1'].reshape(1, -1), lp['We2'], lp['be2'].reshape(1, -1),
            lp['Wx'].reshape(1, -1), lp['bx'].reshape(1, -1))
    specs = [eb(D), eb(D), eb(CP), eb(CP), eb(1)] + [fb(a) for a in args[5:]]
    return pl.pallas_call(
        _edge_body,
        grid=(nb,),
        in_specs=specs,
        out_specs=[eb(D), eb(CP)],
        out_shape=[jax.ShapeDtypeStruct((E, D), _f32),
                   jax.ShapeDtypeStruct((E, CP), _f32)],
    )(*args)


def _node_body(h_ref, cp_ref, a0_ref, a1_ref, c0_ref, c1_ref,
               wa_ref, wb_ref, bh1_ref, w2_ref, bh2_ref,
               hout_ref, cpout_ref):
    agg = a0_ref[...] + a1_ref[...]
    cd = c0_ref[...] + c1_ref[...]
    deg = cd[:, 3:4] + 1.0
    lane = lax.broadcasted_iota(jnp.int32, (N, CP), 1)
    cpout_ref[...] = cp_ref[...] + jnp.where(lane < 3, cd / deg, 0.0)
    z = (jnp.dot(h_ref[...], wa_ref[...], preferred_element_type=_f32)
         + jnp.dot(agg, wb_ref[...], preferred_element_type=_f32)
         + bh1_ref[...])
    hu = z * jax.nn.sigmoid(z)
    hout_ref[...] = (h_ref[...] +
                     jnp.dot(hu, w2_ref[...], preferred_element_type=_f32)
                     + bh2_ref[...])


def _node_call(h, coordp, agg0, agg1, cd0, cd1, lp):
    return pl.pallas_call(
        _node_body,
        out_shape=[jax.ShapeDtypeStruct((N, D), _f32),
                   jax.ShapeDtypeStruct((N, CP), _f32)],
    )(h, coordp, agg0, agg1, cd0, cd1,
      lp['Wh1'][0:D], lp['Wh1'][D:], lp['bh1'].reshape(1, -1),
      lp['Wh2'], lp['bh2'].reshape(1, -1))


def _heads_body(h_ref, cp_ref, wa_ref, ba_ref, wc_ref, bc_ref,
                wm_ref, bm_ref, ww_ref, bw_ref,
                al_ref, cl_ref, co_ref, mm_ref, lw_ref):
    h = h_ref[...]
    al_ref[...] = jnp.dot(h, wa_ref[...], preferred_element_type=_f32) + ba_ref[...]
    cl_ref[...] = jnp.dot(h, wc_ref[...], preferred_element_type=_f32) + bc_ref[...]
    ct = cp_ref[:, 0:3]
    co_ref[...] = ct
    mm = jnp.dot(h, wm_ref[...], preferred_element_type=_f32) + bm_ref[...]
    mm_ref[...] = mm + jnp.concatenate([ct, ct, ct, ct], axis=1)
    zw = jnp.dot(h, ww_ref[...], preferred_element_type=_f32) + bw_ref[...]
    zmax = jnp.max(zw, axis=1, keepdims=True)
    s = zw - zmax
    lw_ref[...] = s - jnp.log(jnp.sum(jnp.exp(s), axis=1, keepdims=True))


def _heads_call(h, coordp, p):
    return pl.pallas_call(
        _heads_body,
        out_shape=[jax.ShapeDtypeStruct((N, 16), _f32),
                   jax.ShapeDtypeStruct((N, 8), _f32),
                   jax.ShapeDtypeStruct((N, 3), _f32),
                   jax.ShapeDtypeStruct((N, 12), _f32),
                   jax.ShapeDtypeStruct((N, 4), _f32)],
    )(h, coordp, p['Wa'], p['ba'].reshape(1, -1), p['Wc'], p['bc'].reshape(1, -1),
      p['Wm'], p['bm'].reshape(1, -1), p['Ww'], p['bw'].reshape(1, -1))


def _gather_stage(h, coordp, row, col):
    return h[row], h[col], coordp[row], coordp[col]


def _scatter_stage(m2, wd, row):
    agg = jax.ops.segment_sum(m2, row, num_segments=N)
    cd = jax.ops.segment_sum(wd, row, num_segments=N)
    z_agg = jnp.zeros_like(agg)
    z_cd = jnp.zeros_like(cd)
    return agg, z_agg, cd, z_cd


def kernel(x, a, c, e, edge_index, batch, t, params):
    p = params
    row = edge_index[0].astype(jnp.int32)
    col = edge_index[1].astype(jnp.int32)
    a_f = a.astype(_f32).reshape(N, 1)
    c_f = c.astype(_f32).reshape(N, 1)
    b_f = batch.astype(_f32).reshape(N, 1)
    e_f = e.astype(_f32).reshape(E, 1)
    t_col = t.reshape(G, 1)
    coordp = jnp.pad(x, ((0, 0), (0, CP - 3)))

    h = _embed_call(a_f, c_f, b_f, t_col, p)
    for l in range(2):
        lp = p['layers'][l]
        hr, hc, cr, cc = _gather_stage(h, coordp, row, col)
        m2, wd = _edge_call(hr, hc, cr, cc, e_f, lp, p['edge_emb'])
        agg0, agg1, cd0, cd1 = _scatter_stage(m2, wd, row)
        h, coordp = _node_call(h, coordp, agg0, agg1, cd0, cd1, lp)

    al, cl, co, mm, lw = _heads_call(h, coordp, p)
    return al, cl, co, mm.reshape(N, 4, 3), lw, h
